# baseline (speedup 1.0000x reference)
import functools

import jax
import jax.numpy as jnp
from jax import lax
from jax.experimental import pallas as pl
from jax.experimental.pallas import tpu as pltpu
from jax.experimental.pallas import tpu_sc as plsc

BATCH = 16384
EMBED_DIM = 32
NUM_CORES = 2
NUM_SUBCORES = 16
NUM_WORKERS = NUM_CORES * NUM_SUBCORES
ROWS_PER_WORKER = BATCH // NUM_WORKERS
LANES = 16
GROUP = 4
NUM_GROUPS = ROWS_PER_WORKER // GROUP
NUM_PAIRS = NUM_GROUPS // 2
IDS_PAD = ROWS_PER_WORKER + LANES


def _fm_body(uid_hbm, iid_hbm, ut_hbm, it_hbm, out_hbm,
             uids_v, iids_v, slab_v, out_v, sem0, sem1):
    wid = lax.axis_index("s") * NUM_CORES + lax.axis_index("c")
    base = wid * ROWS_PER_WORKER

    pltpu.sync_copy(uid_hbm.at[pl.ds(base, ROWS_PER_WORKER)],
                    uids_v.at[pl.ds(0, ROWS_PER_WORKER)])
    pltpu.sync_copy(iid_hbm.at[pl.ds(base, ROWS_PER_WORKER)],
                    iids_v.at[pl.ds(0, ROWS_PER_WORKER)])

    d16a = lax.iota(jnp.int32, LANES)
    d16b = d16a + LANES
    lane = d16a

    def issue(g, par, sem):
        udv = uids_v[pl.ds(g * GROUP, LANES)]
        idv = iids_v[pl.ds(g * GROUP, LANES)]
        for j in range(GROUP):
            ru = udv[j]
            cb = pl.multiple_of(ru - (ru & 127), 128)
            pltpu.async_copy(ut_hbm.at[:, pl.ds(cb, 128)],
                             slab_v.at[par * 2 * GROUP + j], sem)
            ri = idv[j]
            ci = pl.multiple_of(ri - (ri & 127), 128)
            pltpu.async_copy(it_hbm.at[:, pl.ds(ci, 128)],
                             slab_v.at[par * 2 * GROUP + GROUP + j], sem)

    def wait_group(par, sem):
        for j in range(2 * GROUP):
            pltpu.make_async_copy(ut_hbm.at[:, pl.ds(0, 128)],
                                  slab_v.at[par * 2 * GROUP + j], sem).wait()

    def fm(g, par, half, acc):
        udv = uids_v[pl.ds(g * GROUP, LANES)]
        idv = iids_v[pl.ds(g * GROUP, LANES)]
        for j in range(GROUP):
            rum = jnp.full((LANES,), udv[j] & 127, jnp.int32)
            rim = jnp.full((LANES,), idv[j] & 127, jnp.int32)
            zu = jnp.full((LANES,), par * 2 * GROUP + j, jnp.int32)
            zi = jnp.full((LANES,), par * 2 * GROUP + GROUP + j, jnp.int32)
            ua = plsc.load_gather(slab_v, [zu, d16a, rum])
            ub = plsc.load_gather(slab_v, [zu, d16b, rum])
            ia = plsc.load_gather(slab_v, [zi, d16a, rim])
            ib = plsc.load_gather(slab_v, [zi, d16b, rim])
            s = (ua + ia + ua * ia) + (ub + ib + ub * ib)
            acc = jnp.where(lane == half + j, jnp.sum(s), acc)
        return acc

    issue(0, 0, sem0)

    def step(t, acc):
        g0 = 2 * t
        g1 = g0 + 1
        half = (t % 2) * (2 * GROUP)
        issue(g1, 1, sem1)
        wait_group(0, sem0)
        acc = fm(g0, 0, half, acc)

        @pl.when(g0 + 2 < NUM_GROUPS)
        def _():
            issue(g0 + 2, 0, sem0)

        wait_group(1, sem1)
        acc = fm(g1, 1, half + GROUP, acc)

        @pl.when(t % 2 == 1)
        def _():
            out_v[pl.ds((t // 2) * LANES, LANES)] = acc
        return jnp.where(t % 2 == 1, jnp.zeros((LANES,), jnp.float32), acc)

    lax.fori_loop(0, NUM_PAIRS, step, jnp.zeros((LANES,), jnp.float32))

    pltpu.sync_copy(out_v, out_hbm.at[pl.ds(base, ROWS_PER_WORKER)])


def kernel(user_ids, item_ids, user_table, item_table):
    uids = user_ids.astype(jnp.int32)
    iids = item_ids.astype(jnp.int32)
    mesh = plsc.VectorSubcoreMesh(core_axis_name="c", subcore_axis_name="s")
    fm = functools.partial(
        pl.kernel,
        mesh=mesh,
        compiler_params=pltpu.CompilerParams(needs_layout_passes=False,
                                             use_tc_tiling_on_sc=True),
        out_type=jax.ShapeDtypeStruct((BATCH,), jnp.float32),
        scratch_types=[
            pltpu.VMEM((IDS_PAD,), jnp.int32),
            pltpu.VMEM((IDS_PAD,), jnp.int32),
            pltpu.VMEM((4 * GROUP, EMBED_DIM, 128), jnp.float32),
            pltpu.VMEM((ROWS_PER_WORKER,), jnp.float32),
            pltpu.SemaphoreType.DMA,
            pltpu.SemaphoreType.DMA,
        ],
    )(_fm_body)
    out = fm(uids, iids, user_table.T, item_table.T)
    return out.reshape(BATCH, 1)


# speedup vs baseline: 1.0958x; 1.0958x over previous
import functools

import jax
import jax.numpy as jnp
from jax import lax
from jax.experimental import pallas as pl
from jax.experimental.pallas import tpu as pltpu
from jax.experimental.pallas import tpu_sc as plsc

BATCH = 16384
EMBED_DIM = 32
NUM_CORES = 2
NUM_SUBCORES = 16
NUM_WORKERS = NUM_CORES * NUM_SUBCORES
ROWS_PER_WORKER = BATCH // NUM_WORKERS
LANES = 16
GROUP = 2
NUM_GROUPS = ROWS_PER_WORKER // GROUP
NUM_ITERS = NUM_GROUPS // 4
IDS_PAD = ROWS_PER_WORKER + LANES
SLOTS = 2 * GROUP


def _fm_body(uid_hbm, iid_hbm, ut_hbm, it_hbm, out_hbm,
             uids_v, iids_v, slab_v, out_v, sem0, sem1, sem2, sem3):
    wid = lax.axis_index("s") * NUM_CORES + lax.axis_index("c")
    base = wid * ROWS_PER_WORKER
    sems = [sem0, sem1, sem2, sem3]

    pltpu.sync_copy(uid_hbm.at[pl.ds(base, ROWS_PER_WORKER)],
                    uids_v.at[pl.ds(0, ROWS_PER_WORKER)])
    pltpu.sync_copy(iid_hbm.at[pl.ds(base, ROWS_PER_WORKER)],
                    iids_v.at[pl.ds(0, ROWS_PER_WORKER)])

    d16a = lax.iota(jnp.int32, LANES)
    d16b = d16a + LANES
    lane = d16a

    def issue(g, par):
        udv = uids_v[pl.ds(g * GROUP, LANES)]
        idv = iids_v[pl.ds(g * GROUP, LANES)]
        for j in range(GROUP):
            ru = udv[j]
            cb = pl.multiple_of(ru - (ru & 127), 128)
            pltpu.async_copy(ut_hbm.at[:, pl.ds(cb, 128)],
                             slab_v.at[par * SLOTS + j], sems[par])
            ri = idv[j]
            ci = pl.multiple_of(ri - (ri & 127), 128)
            pltpu.async_copy(it_hbm.at[:, pl.ds(ci, 128)],
                             slab_v.at[par * SLOTS + GROUP + j], sems[par])

    def wait_group(par):
        for j in range(SLOTS):
            pltpu.make_async_copy(ut_hbm.at[:, pl.ds(0, 128)],
                                  slab_v.at[par * SLOTS + j], sems[par]).wait()

    def fm(g, par, half, k, acc):
        udv = uids_v[pl.ds(g * GROUP, LANES)]
        idv = iids_v[pl.ds(g * GROUP, LANES)]
        for j in range(GROUP):
            rum = jnp.full((LANES,), udv[j] & 127, jnp.int32)
            rim = jnp.full((LANES,), idv[j] & 127, jnp.int32)
            zu = jnp.full((LANES,), par * SLOTS + j, jnp.int32)
            zi = jnp.full((LANES,), par * SLOTS + GROUP + j, jnp.int32)
            ua = plsc.load_gather(slab_v, [zu, d16a, rum])
            ub = plsc.load_gather(slab_v, [zu, d16b, rum])
            ia = plsc.load_gather(slab_v, [zi, d16a, rim])
            ib = plsc.load_gather(slab_v, [zi, d16b, rim])
            s = (ua + ia + ua * ia) + (ub + ib + ub * ib)
            acc = jnp.where(lane == half + k * GROUP + j, jnp.sum(s), acc)
        return acc

    issue(0, 0)
    issue(1, 1)
    issue(2, 2)

    def step(t, acc):
        g0 = 4 * t
        half = (t % 2) * (4 * GROUP)
        for k in range(4):
            nxt = g0 + k + 3

            @pl.when(nxt < NUM_GROUPS)
            def _():
                issue(nxt, (k + 3) % 4)

            wait_group(k)
            acc = fm(g0 + k, k, half, k, acc)

        @pl.when(t % 2 == 1)
        def _():
            out_v[pl.ds((t // 2) * LANES, LANES)] = acc
        return jnp.where(t % 2 == 1, jnp.zeros((LANES,), jnp.float32), acc)

    lax.fori_loop(0, NUM_ITERS, step, jnp.zeros((LANES,), jnp.float32))

    pltpu.sync_copy(out_v, out_hbm.at[pl.ds(base, ROWS_PER_WORKER)])


def kernel(user_ids, item_ids, user_table, item_table):
    uids = user_ids.astype(jnp.int32)
    iids = item_ids.astype(jnp.int32)
    mesh = plsc.VectorSubcoreMesh(core_axis_name="c", subcore_axis_name="s")
    fm = functools.partial(
        pl.kernel,
        mesh=mesh,
        compiler_params=pltpu.CompilerParams(needs_layout_passes=False,
                                             use_tc_tiling_on_sc=True),
        out_type=jax.ShapeDtypeStruct((BATCH,), jnp.float32),
        scratch_types=[
            pltpu.VMEM((IDS_PAD,), jnp.int32),
            pltpu.VMEM((IDS_PAD,), jnp.int32),
            pltpu.VMEM((4 * SLOTS, EMBED_DIM, 128), jnp.float32),
            pltpu.VMEM((ROWS_PER_WORKER,), jnp.float32),
            pltpu.SemaphoreType.DMA,
            pltpu.SemaphoreType.DMA,
            pltpu.SemaphoreType.DMA,
            pltpu.SemaphoreType.DMA,
        ],
    )(_fm_body)
    out = fm(uids, iids, user_table.T, item_table.T)
    return out.reshape(BATCH, 1)


# speedup vs baseline: 1.1802x; 1.0770x over previous
import functools

import jax
import jax.numpy as jnp
from jax import lax
from jax.experimental import pallas as pl
from jax.experimental.pallas import tpu as pltpu
from jax.experimental.pallas import tpu_sc as plsc

BATCH = 16384
EMBED_DIM = 32
NUM_CORES = 2
NUM_SUBCORES = 16
NUM_WORKERS = NUM_CORES * NUM_SUBCORES
ROWS_PER_WORKER = BATCH // NUM_WORKERS
LANES = 16
GROUP = 1
NUM_GROUPS = ROWS_PER_WORKER // GROUP
NUM_ITERS = NUM_GROUPS // 8
IDS_PAD = ROWS_PER_WORKER + LANES
SLOTS = 2 * GROUP


def _fm_body(uid_hbm, iid_hbm, ut_hbm, it_hbm, out_hbm,
             uids_v, iids_v, slab_v, out_v,
             sem0, sem1, sem2, sem3, sem4, sem5, sem6, sem7):
    wid = lax.axis_index("s") * NUM_CORES + lax.axis_index("c")
    base = wid * ROWS_PER_WORKER
    sems = [sem0, sem1, sem2, sem3, sem4, sem5, sem6, sem7]

    pltpu.sync_copy(uid_hbm.at[pl.ds(base, ROWS_PER_WORKER)],
                    uids_v.at[pl.ds(0, ROWS_PER_WORKER)])
    pltpu.sync_copy(iid_hbm.at[pl.ds(base, ROWS_PER_WORKER)],
                    iids_v.at[pl.ds(0, ROWS_PER_WORKER)])

    d16a = lax.iota(jnp.int32, LANES)
    d16b = d16a + LANES
    lane = d16a

    def issue(g, par):
        udv = uids_v[pl.ds(g * GROUP, LANES)]
        idv = iids_v[pl.ds(g * GROUP, LANES)]
        for j in range(GROUP):
            ru = udv[j]
            cb = pl.multiple_of(ru - (ru & 127), 128)
            pltpu.async_copy(ut_hbm.at[:, pl.ds(cb, 128)],
                             slab_v.at[par * SLOTS + j], sems[par])
            ri = idv[j]
            ci = pl.multiple_of(ri - (ri & 127), 128)
            pltpu.async_copy(it_hbm.at[:, pl.ds(ci, 128)],
                             slab_v.at[par * SLOTS + GROUP + j], sems[par])

    def wait_group(par):
        for j in range(SLOTS):
            pltpu.make_async_copy(ut_hbm.at[:, pl.ds(0, 128)],
                                  slab_v.at[par * SLOTS + j], sems[par]).wait()

    def fm(g, par, half, k, acc):
        udv = uids_v[pl.ds(g * GROUP, LANES)]
        idv = iids_v[pl.ds(g * GROUP, LANES)]
        for j in range(GROUP):
            rum = jnp.full((LANES,), udv[j] & 127, jnp.int32)
            rim = jnp.full((LANES,), idv[j] & 127, jnp.int32)
            zu = jnp.full((LANES,), par * SLOTS + j, jnp.int32)
            zi = jnp.full((LANES,), par * SLOTS + GROUP + j, jnp.int32)
            ua = plsc.load_gather(slab_v, [zu, d16a, rum])
            ub = plsc.load_gather(slab_v, [zu, d16b, rum])
            ia = plsc.load_gather(slab_v, [zi, d16a, rim])
            ib = plsc.load_gather(slab_v, [zi, d16b, rim])
            s = (ua + ia + ua * ia) + (ub + ib + ub * ib)
            acc = jnp.where(lane == half + k * GROUP + j, jnp.sum(s), acc)
        return acc

    for p in range(7):
        issue(p, p)

    def step(t, acc):
        g0 = 8 * t
        half = (t % 2) * (8 * GROUP)
        for k in range(8):
            nxt = g0 + k + 7

            @pl.when(nxt < NUM_GROUPS)
            def _():
                issue(nxt, (k + 7) % 8)

            wait_group(k)
            acc = fm(g0 + k, k, half, k, acc)

        @pl.when(t % 2 == 1)
        def _():
            out_v[pl.ds((t // 2) * LANES, LANES)] = acc
        return jnp.where(t % 2 == 1, jnp.zeros((LANES,), jnp.float32), acc)

    lax.fori_loop(0, NUM_ITERS, step, jnp.zeros((LANES,), jnp.float32))

    pltpu.sync_copy(out_v, out_hbm.at[pl.ds(base, ROWS_PER_WORKER)])


def kernel(user_ids, item_ids, user_table, item_table):
    uids = user_ids.astype(jnp.int32)
    iids = item_ids.astype(jnp.int32)
    mesh = plsc.VectorSubcoreMesh(core_axis_name="c", subcore_axis_name="s")
    fm = functools.partial(
        pl.kernel,
        mesh=mesh,
        compiler_params=pltpu.CompilerParams(needs_layout_passes=False,
                                             use_tc_tiling_on_sc=True),
        out_type=jax.ShapeDtypeStruct((BATCH,), jnp.float32),
        scratch_types=[
            pltpu.VMEM((IDS_PAD,), jnp.int32),
            pltpu.VMEM((IDS_PAD,), jnp.int32),
            pltpu.VMEM((8 * SLOTS, EMBED_DIM, 128), jnp.float32),
            pltpu.VMEM((ROWS_PER_WORKER,), jnp.float32),
            pltpu.SemaphoreType.DMA,
            pltpu.SemaphoreType.DMA,
            pltpu.SemaphoreType.DMA,
            pltpu.SemaphoreType.DMA,
            pltpu.SemaphoreType.DMA,
            pltpu.SemaphoreType.DMA,
            pltpu.SemaphoreType.DMA,
            pltpu.SemaphoreType.DMA,
        ],
    )(_fm_body)
    out = fm(uids, iids, user_table.T, item_table.T)
    return out.reshape(BATCH, 1)
